# baseline (speedup 1.0000x reference)
import functools

import jax
import jax.numpy as jnp
from jax import lax
from jax.experimental import pallas as pl
from jax.experimental.pallas import tpu as pltpu
from jax.experimental.pallas import tpu_sc as plsc

_NC = 2
_NS = 16
_NW = _NC * _NS
_L = 16


def _sc_body(Tpe, B, D, RC, x_hbm, pe_hbm, out_hbm, pe_buf, x_buf):
    w = lax.axis_index("s") * _NC + lax.axis_index("c")
    tpw = Tpe // _NW
    t0 = w * tpw

    def chunk(k, carry):
        tk = t0 + k * RC
        pltpu.sync_copy(pe_hbm.at[pl.ds(tk, RC)], pe_buf)

        def bstep(b, c2):
            row = b * Tpe + tk
            pltpu.sync_copy(x_hbm.at[pl.ds(row, RC)], x_buf)

            def radd(r, c3):
                for c in range(D // _L):
                    plsc.addupdate(
                        x_buf.at[r, pl.ds(c * _L, _L)],
                        pe_buf[r, pl.ds(c * _L, _L)],
                    )
                return c3

            lax.fori_loop(0, RC, radd, 0)
            pltpu.sync_copy(x_buf, out_hbm.at[pl.ds(row, RC)])
            return c2

        lax.fori_loop(0, B, bstep, 0)
        return carry

    lax.fori_loop(0, tpw // RC, chunk, 0)


def kernel(x, pe, T):
    del T
    B, S, D = x.shape
    Tpe = pe.shape[0]
    RC = 32

    mesh = plsc.VectorSubcoreMesh(core_axis_name="c", subcore_axis_name="s")
    sc_add = functools.partial(
        pl.kernel,
        out_type=jax.ShapeDtypeStruct((B * S, D), jnp.float32),
        mesh=mesh,
        scratch_types=[
            pltpu.VMEM((RC, D), jnp.float32),
            pltpu.VMEM((RC, D), jnp.float32),
        ],
    )(functools.partial(_sc_body, Tpe, B, D, RC))

    out2d = sc_add(x.reshape(B * S, D), pe)
    return out2d.reshape(B, S, D)


# speedup vs baseline: 1.6162x; 1.6162x over previous
import functools

import jax
import jax.numpy as jnp
from jax import lax
from jax.experimental import pallas as pl
from jax.experimental.pallas import tpu as pltpu
from jax.experimental.pallas import tpu_sc as plsc

_NC = 2
_NS = 16
_NW = _NC * _NS
_L = 16


def _tc_body(x_ref, pe_ref, o_ref):
    o_ref[...] = x_ref[...] + pe_ref[...][None, :, :]


def _sc_body(Tpe, row0, RC, D, x_hbm, pe_hbm, out_hbm, pe_buf, x_buf):
    w = lax.axis_index("s") * _NC + lax.axis_index("c")
    tpw = Tpe // _NW
    t0 = w * tpw

    def chunk(k, carry):
        tk = t0 + k * RC
        pltpu.sync_copy(pe_hbm.at[pl.ds(tk, RC)], pe_buf)
        pltpu.sync_copy(x_hbm.at[pl.ds(row0 + tk, RC)], x_buf)

        def radd(r, c3):
            for c in range(D // _L):
                plsc.addupdate(
                    x_buf.at[r, pl.ds(c * _L, _L)],
                    pe_buf[r, pl.ds(c * _L, _L)],
                )
            return c3

        lax.fori_loop(0, RC, radd, 0)
        pltpu.sync_copy(x_buf, out_hbm.at[pl.ds(tk, RC)])
        return carry

    lax.fori_loop(0, tpw // RC, chunk, 0)


def kernel(x, pe, T):
    del T
    B, S, D = x.shape
    Tpe = pe.shape[0]
    B_tc = B - 1
    blk = 512
    RC = 32

    tc_out = pl.pallas_call(
        _tc_body,
        grid=(S // blk,),
        in_specs=[
            pl.BlockSpec((B_tc, blk, D), lambda i: (0, i, 0)),
            pl.BlockSpec((blk, D), lambda i: (i, 0)),
        ],
        out_specs=pl.BlockSpec((B_tc, blk, D), lambda i: (0, i, 0)),
        out_shape=jax.ShapeDtypeStruct((B_tc, S, D), x.dtype),
    )(x, pe)

    mesh = plsc.VectorSubcoreMesh(core_axis_name="c", subcore_axis_name="s")
    sc_add = functools.partial(
        pl.kernel,
        out_type=jax.ShapeDtypeStruct((S, D), jnp.float32),
        mesh=mesh,
        scratch_types=[
            pltpu.VMEM((RC, D), jnp.float32),
            pltpu.VMEM((RC, D), jnp.float32),
        ],
    )(functools.partial(_sc_body, Tpe, (B - 1) * S, RC, D))
    sc_out = sc_add(x.reshape(B * S, D), pe)

    return jnp.concatenate([tc_out, sc_out[None]], axis=0)
